# baseline (speedup 1.0000x reference)
import functools

import jax
import jax.numpy as jnp
from jax import lax
from jax.experimental import pallas as pl
from jax.experimental.pallas import tpu as pltpu
from jax.experimental.pallas import tpu_sc as plsc

NC, NS, L = 2, 16, 16
B, S, T, D = 16, 9, 1024, 256
N = B * S * T * D

CH = 262144
HALF = N // NC
NCH = HALF // CH


def _sc_body(x_hbm, w_hbm, o_hbm, spbuf, s0, s1, so):
    cid = lax.axis_index("c")
    sid = lax.axis_index("s")
    sems = (s0, s1)

    @pl.when(sid == 0)
    def _():
        for b in range(2):
            pltpu.make_async_copy(
                x_hbm.at[pl.ds(cid * HALF + b * CH, CH)],
                spbuf.at[pl.ds(b * CH, CH)], sems[b]
            ).start()

        def lp(o, c):
            for b in range(2):
                i = o * 2 + b
                pltpu.make_async_copy(
                    x_hbm.at[pl.ds(cid * HALF + i * CH, CH)],
                    spbuf.at[pl.ds(b * CH, CH)], sems[b]
                ).wait()

                @pl.when(i + 2 < NCH)
                def _():
                    pltpu.make_async_copy(
                        x_hbm.at[pl.ds(cid * HALF + (i + 2) * CH, CH)],
                        spbuf.at[pl.ds(b * CH, CH)], sems[b]
                    ).start()
            return c

        lax.fori_loop(0, NCH // 2, lp, 0)

        pltpu.make_async_copy(
            spbuf.at[pl.ds(0, CH)], o_hbm.at[pl.ds(cid * HALF, CH)], so
        ).start()
        pltpu.make_async_copy(
            spbuf.at[pl.ds(0, CH)], o_hbm.at[pl.ds(cid * HALF, CH)], so
        ).wait()


_sc_kernel = functools.partial(
    pl.kernel,
    out_type=jax.ShapeDtypeStruct((N,), jnp.float32),
    mesh=plsc.VectorSubcoreMesh(
        core_axis_name="c", subcore_axis_name="s",
        num_cores=NC, num_subcores=NS,
    ),
    scratch_types=[
        pltpu.VMEM_SHARED((2 * CH,), jnp.float32),
        pltpu.SemaphoreType.DMA,
        pltpu.SemaphoreType.DMA,
        pltpu.SemaphoreType.DMA,
    ],
)(_sc_body)


def kernel(x, W):
    out = _sc_kernel(x.reshape(-1), W.reshape(-1))
    return out.reshape(x.shape)


# speedup vs baseline: 3.2731x; 3.2731x over previous
import functools

import jax
import jax.numpy as jnp
from jax import lax
from jax.experimental import pallas as pl
from jax.experimental.pallas import tpu as pltpu
from jax.experimental.pallas import tpu_sc as plsc

NC, NS, L = 2, 16, 16
S, D = 9, 256


def _rsqrt(v):
    i = lax.bitcast_convert_type(v, jnp.int32)
    i = jnp.int32(0x5F3759DF) - lax.shift_right_arithmetic(i, 1)
    y = lax.bitcast_convert_type(i, jnp.float32)
    for _ in range(3):
        y = y * (1.5 - 0.5 * v * y * y)
    return y


def _norm_body(w_hbm, emb_hbm, w_vmem, emb_vmem, fold_vmem):
    cid = lax.axis_index("c")
    sid = lax.axis_index("s")

    @pl.when((cid == 0) & (sid == 0))
    def _():
        pltpu.sync_copy(w_hbm, w_vmem)
        for r in range(S):
            wr = [w_vmem[pl.ds(r * D + k * L, L)] for k in range(D // L)]
            acc = wr[0] * wr[0]
            for k in range(1, D // L):
                acc = acc + wr[k] * wr[k]
            fold_vmem[pl.ds(0, L)] = acc
            fold_vmem[pl.ds(L, L)] = jnp.zeros((L,), jnp.float32)
            for sh in (8, 4, 2, 1):
                a = fold_vmem[pl.ds(0, L)]
                shifted = fold_vmem[pl.ds(sh, L)]
                fold_vmem[pl.ds(0, L)] = a + shifted
            n2 = jnp.full((L,), fold_vmem[pl.ds(0, L)][0], jnp.float32)
            norm = n2 * _rsqrt(n2)
            scale = jnp.where(n2 > 1.0, 1.0 / (norm + 1e-7), jnp.float32(1.0))
            for k in range(D // L):
                emb_vmem[pl.ds(r * D + k * L, L)] = wr[k] * scale
        pltpu.sync_copy(emb_vmem, emb_hbm)


_sc_norm = functools.partial(
    pl.kernel,
    out_type=jax.ShapeDtypeStruct((S * D,), jnp.float32),
    mesh=plsc.VectorSubcoreMesh(
        core_axis_name="c", subcore_axis_name="s",
        num_cores=NC, num_subcores=NS,
    ),
    scratch_types=[
        pltpu.VMEM((S * D,), jnp.float32),
        pltpu.VMEM((S * D,), jnp.float32),
        pltpu.VMEM((2 * L,), jnp.float32),
    ],
)(_norm_body)


def _add_body(x_ref, emb_ref, o_ref):
    o_ref[...] = x_ref[...] + emb_ref[...][None, :, None, :]


def kernel(x, W):
    B, S_, T, D_ = x.shape
    emb = _sc_norm(W.reshape(-1)).reshape(S_, D_)
    return pl.pallas_call(
        _add_body,
        grid=(B,),
        in_specs=[
            pl.BlockSpec((1, S_, T, D_), lambda i: (i, 0, 0, 0)),
            pl.BlockSpec((S_, D_), lambda i: (0, 0)),
        ],
        out_specs=pl.BlockSpec((1, S_, T, D_), lambda i: (i, 0, 0, 0)),
        out_shape=jax.ShapeDtypeStruct(x.shape, x.dtype),
    )(x, emb)


# speedup vs baseline: 4.0639x; 1.2416x over previous
import jax
import jax.numpy as jnp
from jax.experimental import pallas as pl


def _body(x_ref, w_ref, o_ref):
    w = w_ref[...]
    norms = jnp.sqrt(jnp.sum(w * w, axis=1, keepdims=True))
    emb = jnp.where(norms > 1.0, w * (1.0 / (norms + 1e-7)), w)
    o_ref[...] = x_ref[...] + emb[None, :, None, :]


def kernel(x, W):
    B, S, T, D = x.shape
    return pl.pallas_call(
        _body,
        grid=(B,),
        in_specs=[
            pl.BlockSpec((1, S, T, D), lambda i: (i, 0, 0, 0)),
            pl.BlockSpec((S, D), lambda i: (0, 0)),
        ],
        out_specs=pl.BlockSpec((1, S, T, D), lambda i: (i, 0, 0, 0)),
        out_shape=jax.ShapeDtypeStruct(x.shape, x.dtype),
    )(x, W)
